# baseline (speedup 1.0000x reference)
import functools

import jax
import jax.numpy as jnp
from jax.experimental import pallas as pl
from jax.experimental.pallas import tpu as pltpu
from jax.experimental.pallas import tpu_sc as plsc

EMB = 32
WINDOW = 1600


def kernel(x, T0, T1, T2, T3):
    F, B, S = x.shape
    N = B * S
    M = F * N

    tcat = jnp.concatenate([T0, T1, T2, T3], axis=0)
    off = (jnp.arange(F, dtype=jnp.int32) * T0.shape[0])[:, None, None]
    idx = (x.astype(jnp.int32) + off).transpose(1, 2, 0).reshape(1, M)

    mesh = plsc.VectorSubcoreMesh(
        core_axis_name="core", subcore_axis_name="subcore"
    )

    @functools.partial(
        pl.kernel,
        out_type=jax.ShapeDtypeStruct((M, EMB), jnp.float32),
        mesh=mesh,
        compiler_params=pltpu.CompilerParams(use_tc_tiling_on_sc=False),
    )
    def sc_gather(i_hbm, t_hbm, out_hbm):
        def body(i_vmem, o_vmem):
            pltpu.sync_copy(t_hbm.at[i_vmem.at[0]], o_vmem)

        pltpu.emit_pipeline(
            body,
            grid=(M // WINDOW,),
            in_specs=[pl.BlockSpec((1, WINDOW), index_map=lambda j: (0, j))],
            out_specs=[pl.BlockSpec((WINDOW, EMB), index_map=lambda j: (j, 0))],
            core_axis_name=("core", "subcore"),
            dimension_semantics=(pltpu.PARALLEL,),
        )(i_hbm, out_hbm)

    out = sc_gather(idx, tcat)
    return out.reshape(B, S, F * EMB)


# speedup vs baseline: 3.0868x; 3.0868x over previous
import functools

import jax
import jax.numpy as jnp
from jax.experimental import pallas as pl
from jax.experimental.pallas import tpu as pltpu
from jax.experimental.pallas import tpu_sc as plsc

EMB = 32
N_FIELDS = 4
ROWS = 8
L = 16


def kernel(x, T0, T1, T2, T3):
    F, B, S = x.shape
    N = B * S
    W = ROWS * S
    x = x.astype(jnp.int32)

    mesh = plsc.VectorSubcoreMesh(
        core_axis_name="core", subcore_axis_name="subcore"
    )

    starts = list(range(0, S - L + 1, L))
    if starts[-1] != S - L:
        starts.append(S - L)

    @functools.partial(
        pl.kernel,
        out_type=jax.ShapeDtypeStruct((N, N_FIELDS * EMB), jnp.float32),
        mesh=mesh,
        compiler_params=pltpu.CompilerParams(use_tc_tiling_on_sc=False),
        scratch_types=[pltpu.VMEM((W,), jnp.int32)],
    )
    def sc_gather(x_hbm, t0, t1, t2, t3, out_hbm, flat_idx):
        tables = [t0, t1, t2, t3]
        for f in range(N_FIELDS):
            table = tables[f]

            def body(i_vmem, o_vmem, table=table):
                for r in range(ROWS):
                    for c in starts:
                        flat_idx[pl.ds(r * S + c, L)] = i_vmem[0, r, pl.ds(c, L)]
                pltpu.sync_copy(table.at[flat_idx], o_vmem)

            pltpu.emit_pipeline(
                body,
                grid=(B // ROWS,),
                in_specs=[
                    pl.BlockSpec((1, ROWS, S), index_map=lambda j, f=f: (f, j, 0))
                ],
                out_specs=[
                    pl.BlockSpec((W, EMB), index_map=lambda j, f=f: (j, f))
                ],
                core_axis_name=("core", "subcore"),
                dimension_semantics=(pltpu.PARALLEL,),
            )(x_hbm, out_hbm)

    out = sc_gather(x, T0, T1, T2, T3)
    return out.reshape(B, S, N_FIELDS * EMB)
